# baseline (speedup 1.0000x reference)
import functools

import jax
import jax.numpy as jnp
from jax import lax
from jax.experimental import pallas as pl
from jax.experimental.pallas import tpu as pltpu
from jax.experimental.pallas import tpu_sc as plsc

_L = 4096
_D = 1024
_NC = 2
_NS = 16
_NW = _NC * _NS
_ROWS_PER_W = _L // _NW


def _make_copy_kernel():
    mesh = plsc.VectorSubcoreMesh(core_axis_name="c", subcore_axis_name="s")

    @functools.partial(
        pl.kernel,
        mesh=mesh,
        out_type=jax.ShapeDtypeStruct((_L, _D), jnp.float32),
    )
    def copy_k(pe_hbm, out_hbm):
        wid = lax.axis_index("s") * _NC + lax.axis_index("c")
        base = wid * _ROWS_PER_W
        pltpu.sync_copy(
            pe_hbm.at[pl.ds(base, _ROWS_PER_W)],
            out_hbm.at[pl.ds(base, _ROWS_PER_W)],
        )

    return copy_k


_copy_kernel = _make_copy_kernel()


def kernel(x, pe):
    out = _copy_kernel(pe)
    return out[None]


# speedup vs baseline: 16.6263x; 16.6263x over previous
import functools

import jax
import jax.numpy as jnp
from jax import lax
from jax.experimental import pallas as pl
from jax.experimental.pallas import tpu as pltpu
from jax.experimental.pallas import tpu_sc as plsc

_L = 4096
_D = 1024
_NC = 2
_NS = 16
_NW = _NC * _NS
_ROWS_PER_W = _L // _NW
_CHUNK = 16
_NCHUNK = _ROWS_PER_W // _CHUNK
_NBUF = 4


def _make_copy_kernel():
    mesh = plsc.VectorSubcoreMesh(core_axis_name="c", subcore_axis_name="s")

    @functools.partial(
        pl.kernel,
        mesh=mesh,
        out_type=jax.ShapeDtypeStruct((_L, _D), jnp.float32),
        scratch_types=(
            [pltpu.VMEM((_NBUF, _CHUNK, _D), jnp.float32)]
            + [pltpu.SemaphoreType.DMA] * (2 * _NBUF)
        ),
    )
    def copy_k(pe_hbm, out_hbm, buf, *sems):
        in_sems = sems[:_NBUF]
        out_sems = sems[_NBUF:]
        wid = lax.axis_index("s") * _NC + lax.axis_index("c")
        base = wid * _ROWS_PER_W

        def in_copy(c):
            b = c % _NBUF
            return pltpu.make_async_copy(
                pe_hbm.at[pl.ds(base + c * _CHUNK, _CHUNK)], buf.at[b], in_sems[b]
            )

        def out_copy(c):
            b = c % _NBUF
            return pltpu.make_async_copy(
                buf.at[b], out_hbm.at[pl.ds(base + c * _CHUNK, _CHUNK)], out_sems[b]
            )

        outs = [None] * _NCHUNK
        ins = [None] * _NCHUNK
        for c in range(min(_NBUF, _NCHUNK)):
            ins[c] = in_copy(c)
            ins[c].start()
        for c in range(_NCHUNK):
            ins[c].wait()
            outs[c] = out_copy(c)
            outs[c].start()
            nc = c + _NBUF
            if nc < _NCHUNK:
                outs[c].wait()
                ins[nc] = in_copy(nc)
                ins[nc].start()
        for c in range(max(_NCHUNK - _NBUF, 0), _NCHUNK):
            outs[c].wait()

    return copy_k


_copy_kernel = _make_copy_kernel()


def kernel(x, pe):
    out = _copy_kernel(pe)
    return out[None]
